# baseline (speedup 1.0000x reference)
import functools

import jax
import jax.numpy as jnp
from jax import lax
from jax.experimental import pallas as pl
from jax.experimental.pallas import tpu as pltpu
from jax.experimental.pallas import tpu_sc as plsc

BATCH = 16384
IN_F = 512
OUT_F = 64

NC = 2
NS = 16
L = 16
NW = NC * NS
ROWS_W = BATCH // NW
CHUNK = 64
NCHUNK = ROWS_W // CHUNK
OCHUNK = 128
NOC = ROWS_W // OCHUNK
NG = OUT_F // L


def _sc_gather(x, columns):
    mesh = plsc.VectorSubcoreMesh(core_axis_name="c", subcore_axis_name="s")

    @functools.partial(
        pl.kernel,
        mesh=mesh,
        out_type=jax.ShapeDtypeStruct((OUT_F, BATCH), jnp.float32),
        compiler_params=pltpu.CompilerParams(
            needs_layout_passes=False,
            skip_device_barrier=True,
        ),
        scratch_types=[
            pltpu.VMEM((OUT_F,), jnp.int32),
            pltpu.VMEM((CHUNK, IN_F), jnp.float32),
            pltpu.VMEM((CHUNK, IN_F), jnp.float32),
            pltpu.VMEM((OUT_F, OCHUNK), jnp.float32),
            pltpu.VMEM((OUT_F, OCHUNK), jnp.float32),
            pltpu.SemaphoreType.DMA,
            pltpu.SemaphoreType.DMA,
            pltpu.SemaphoreType.DMA,
            pltpu.SemaphoreType.DMA,
        ],
    )
    def k(x_hbm, cols_hbm, out_hbm, cols_v, in0, in1, ou0, ou1, is0, is1, os0, os1):
        wid = lax.axis_index("s") * NC + lax.axis_index("c")
        base = wid * ROWS_W
        pltpu.sync_copy(cols_hbm, cols_v)
        col_regs = [cols_v[pl.ds(g * L, L)] for g in range(NG)]
        out_cols = [lax.iota(jnp.int32, L) + g * L for g in range(NG)]
        ins = [in0, in1]
        isem = [is0, is1]

        def start_load(ci, slot):
            row0 = base + (ci % NCHUNK) * CHUNK
            return pltpu.async_copy(
                x_hbm.at[pl.ds(row0, CHUNK)], ins[slot], isem[slot]
            )

        def compute_half(ib, ob, h):
            @plsc.parallel_loop(0, CHUNK, unroll=4)
            def row_body(r):
                ridx = jnp.zeros((L,), jnp.int32) + r
                cidx = ridx + h * CHUNK
                for g in range(NG):
                    vals = plsc.load_gather(ib, [ridx, col_regs[g]])
                    plsc.store_scatter(ob, [out_cols[g], cidx], vals)

        start_load(0, 0).wait()

        def group_body(t, _):
            ci0 = 4 * t
            ld = start_load(ci0 + 1, 1)
            compute_half(ins[0], ou0, 0)
            ld.wait()
            ld = start_load(ci0 + 2, 0)
            compute_half(ins[1], ou0, 1)
            st0 = pltpu.async_copy(
                ou0, out_hbm.at[:, pl.ds(base + 2 * t * OCHUNK, OCHUNK)], os0
            )
            ld.wait()
            ld = start_load(ci0 + 3, 1)
            compute_half(ins[0], ou1, 0)
            ld.wait()
            ld = start_load(ci0 + 4, 0)
            compute_half(ins[1], ou1, 1)
            st1 = pltpu.async_copy(
                ou1, out_hbm.at[:, pl.ds(base + (2 * t + 1) * OCHUNK, OCHUNK)], os1
            )
            st0.wait()
            st1.wait()
            ld.wait()
            return 0

        lax.fori_loop(0, NOC // 2, group_body, 0)

    return k(x, columns)


def kernel(x, columns):
    return _sc_gather(x, columns).T
